# baseline (speedup 1.0000x reference)
import functools

import jax
import jax.numpy as jnp
from jax import lax
from jax.experimental import pallas as pl
from jax.experimental.pallas import tpu as pltpu
from jax.experimental.pallas import tpu_sc as plsc

_HEADS = 4
_C = 128
_N = 10000
_G = 512
_P = 64
_SLOP = 0.2

_NC = 2
_NS = 16
_NW = _NC * _NS
_GPW = _G // _NW
_IPG = 2 * 2 * _P


def _tc_body(x_ref, w_ref, att_ref, a_ref):
    w = w_ref[...]
    att2 = att_ref[...].reshape(_HEADS, 2 * _C)
    dn = (((1,), (0,)), ((), ()))
    hp = lax.Precision.HIGHEST

    def vrow(h, half):
        avec = att2[h:h + 1, half * _C:(half + 1) * _C]
        wblk = w[h * _C:(h + 1) * _C, :]
        return lax.dot_general(avec, wblk, dn, precision=hp)

    order = [(0, 0), (2, 0), (0, 1), (2, 1), (1, 0), (3, 0), (1, 1), (3, 1)]
    vcat = jnp.concatenate([vrow(h, half) for h, half in order], axis=0)
    dnx = (((1,), (1,)), ((), ()))
    a2 = lax.dot_general(x_ref[...], vcat, dnx)
    lo = lax.bitcast_convert_type(a2[:, :4].astype(jnp.bfloat16), jnp.uint16)
    hi = lax.bitcast_convert_type(a2[:, 4:].astype(jnp.bfloat16), jnp.uint16)
    a_ref[...] = jnp.bitwise_or(
        lo.astype(jnp.int32),
        jnp.left_shift(hi.astype(jnp.int32), 16))


def _leaky(s):
    return jnp.maximum(s, s * _SLOP)


@functools.cache
def _make_sc_kernel():
    mesh = plsc.VectorSubcoreMesh(core_axis_name="c", subcore_axis_name="s")
    return functools.partial(
        pl.kernel,
        mesh=mesh,
        compiler_params=pltpu.CompilerParams(
            needs_layout_passes=False,
            skip_device_barrier=True,
            disable_bounds_checks=True,
            disable_semaphore_checks=True,
        ),
        out_type=jax.ShapeDtypeStruct((_G * 16,), jnp.float32),
        scratch_types=[
            pltpu.VMEM((_N * _HEADS,), jnp.int32),
            pltpu.VMEM((_GPW * _IPG,), jnp.int32),
            pltpu.VMEM((256,), jnp.float32),
            pltpu.VMEM((16,), jnp.float32),
            pltpu.VMEM((_GPW * 16,), jnp.float32),
        ],
    )(_sc_body)


def _unpack_pair(w):
    lo = plsc.bitcast(jnp.left_shift(w, 16), jnp.float32)
    hi = plsc.bitcast(jnp.bitwise_and(w, jnp.int32(-65536)), jnp.float32)
    return lo, hi


def _sc_body(a_hbm, idx_hbm, out_hbm, a_v, idx_v, tr_v, sm_v, out_v):
    wid = lax.axis_index("s") * _NC + lax.axis_index("c")
    pltpu.sync_copy(a_hbm, a_v)
    pltpu.sync_copy(idx_hbm.at[pl.ds(wid * (_GPW * _IPG), _GPW * _IPG)], idx_v)

    lane = lax.iota(jnp.int32, 16)
    perm_j = jnp.bitwise_xor(lane, 4)

    def group_body(g, carry):
        gb = g * _IPG
        kidx = [[idx_v[pl.ds(gb + i * 2 * _P + _P + c4 * 16, 16)] * _HEADS
                 for c4 in range(4)] for i in range(2)]
        vidx = [[idx_v[pl.ds(gb + j * 2 * _P + c4 * 16, 16)] * _HEADS
                 for c4 in range(4)] for j in range(2)]
        akv = {}
        avv = {}
        for c4 in range(4):
            for hp in range(2):
                for i in range(2):
                    w = plsc.load_gather(a_v, [kidx[i][c4] + hp])
                    akv[i, 2 * hp, c4], akv[i, 2 * hp + 1, c4] = _unpack_pair(w)
                for j in range(2):
                    w = plsc.load_gather(a_v, [vidx[j][c4] + (2 + hp)])
                    avv[j, 2 * hp, c4], avv[j, 2 * hp + 1, c4] = _unpack_pair(w)
        for i in range(2):
            for j in range(2):
                for h in range(_HEADS):
                    q = i * 8 + j * 4 + h
                    acc = _leaky(akv[i, h, 0] + avv[j, h, 0])
                    for c4 in range(1, 4):
                        acc = acc + _leaky(akv[i, h, c4] + avv[j, h, c4])
                    tr_v[pl.ds(q * 16, 16)] = acc
        s = plsc.load_gather(tr_v, [lane * 16])
        for l in range(1, 16):
            s = s + plsc.load_gather(tr_v, [lane * 16 + l])
        s = s * (1.0 / _P)
        sm_v[...] = s
        s_sw = plsc.load_gather(sm_v, [perm_j])
        m = jnp.maximum(s, s_sw)
        e = jnp.exp(s - m)
        sm_v[...] = e
        e_sw = plsc.load_gather(sm_v, [perm_j])
        out_v[pl.ds(g * 16, 16)] = e / (e + e_sw)
        return carry

    lax.fori_loop(0, _GPW, group_body, 0, unroll=2)
    pltpu.sync_copy(out_v, out_hbm.at[pl.ds(wid * (_GPW * 16), _GPW * 16)])


def kernel(x, edge_index, node_idxes, W, att):
    del edge_index
    a_packed = pl.pallas_call(
        _tc_body,
        out_shape=jax.ShapeDtypeStruct((_N, _HEADS), jnp.int32),
    )(x, W, att)
    idx_flat = node_idxes.reshape(_G * _IPG).astype(jnp.int32)
    out = _make_sc_kernel()(a_packed.reshape(_N * _HEADS), idx_flat)
    return out.reshape(_G, 2, 2, _HEADS)


# speedup vs baseline: 1.0427x; 1.0427x over previous
import functools

import jax
import jax.numpy as jnp
from jax import lax
from jax.experimental import pallas as pl
from jax.experimental.pallas import tpu as pltpu
from jax.experimental.pallas import tpu_sc as plsc

_HEADS = 4
_C = 128
_N = 10000
_G = 512
_P = 64
_SLOP = 0.2

_NC = 1
_NS = 16
_NW = _NC * _NS
_GPW = _G // _NW
_IPG = 2 * 2 * _P


def _tc_body(x_ref, w_ref, att_ref, a_ref):
    w = w_ref[...]
    att2 = att_ref[...].reshape(_HEADS, 2 * _C)
    dn = (((1,), (0,)), ((), ()))
    hp = lax.Precision.HIGHEST

    def vrow(h, half):
        avec = att2[h:h + 1, half * _C:(half + 1) * _C]
        wblk = w[h * _C:(h + 1) * _C, :]
        return lax.dot_general(avec, wblk, dn, precision=hp)

    order = [(0, 0), (2, 0), (0, 1), (2, 1), (1, 0), (3, 0), (1, 1), (3, 1)]
    vcat = jnp.concatenate([vrow(h, half) for h, half in order], axis=0)
    dnx = (((1,), (1,)), ((), ()))
    a2 = lax.dot_general(x_ref[...], vcat, dnx)
    lo = lax.bitcast_convert_type(a2[:, :4].astype(jnp.bfloat16), jnp.uint16)
    hi = lax.bitcast_convert_type(a2[:, 4:].astype(jnp.bfloat16), jnp.uint16)
    a_ref[...] = jnp.bitwise_or(
        lo.astype(jnp.int32),
        jnp.left_shift(hi.astype(jnp.int32), 16))


def _leaky(s):
    return jnp.maximum(s, s * _SLOP)


@functools.cache
def _make_sc_kernel():
    mesh = plsc.VectorSubcoreMesh(core_axis_name="c", subcore_axis_name="s", num_cores=_NC)
    return functools.partial(
        pl.kernel,
        mesh=mesh,
        compiler_params=pltpu.CompilerParams(
            needs_layout_passes=False,
            skip_device_barrier=True,
            disable_bounds_checks=True,
            disable_semaphore_checks=True,
        ),
        out_type=jax.ShapeDtypeStruct((_G * 16,), jnp.float32),
        scratch_types=[
            pltpu.VMEM((_N * _HEADS,), jnp.int32),
            pltpu.VMEM((_GPW * _IPG,), jnp.int32),
            pltpu.VMEM((256,), jnp.float32),
            pltpu.VMEM((16,), jnp.float32),
            pltpu.VMEM((_GPW * 16,), jnp.float32),
        ],
    )(_sc_body)


def _unpack_pair(w):
    lo = plsc.bitcast(jnp.left_shift(w, 16), jnp.float32)
    hi = plsc.bitcast(jnp.bitwise_and(w, jnp.int32(-65536)), jnp.float32)
    return lo, hi


def _sc_body(a_hbm, idx_hbm, out_hbm, a_v, idx_v, tr_v, sm_v, out_v):
    wid = lax.axis_index("s") * _NC + lax.axis_index("c")
    pltpu.sync_copy(a_hbm, a_v)
    pltpu.sync_copy(idx_hbm.at[pl.ds(wid * (_GPW * _IPG), _GPW * _IPG)], idx_v)

    lane = lax.iota(jnp.int32, 16)
    perm_j = jnp.bitwise_xor(lane, 4)

    def group_body(g, carry):
        gb = g * _IPG
        kidx = [[idx_v[pl.ds(gb + i * 2 * _P + _P + c4 * 16, 16)] * _HEADS
                 for c4 in range(4)] for i in range(2)]
        vidx = [[idx_v[pl.ds(gb + j * 2 * _P + c4 * 16, 16)] * _HEADS
                 for c4 in range(4)] for j in range(2)]
        akv = {}
        avv = {}
        for c4 in range(4):
            for hp in range(2):
                for i in range(2):
                    w = plsc.load_gather(a_v, [kidx[i][c4] + hp])
                    akv[i, 2 * hp, c4], akv[i, 2 * hp + 1, c4] = _unpack_pair(w)
                for j in range(2):
                    w = plsc.load_gather(a_v, [vidx[j][c4] + (2 + hp)])
                    avv[j, 2 * hp, c4], avv[j, 2 * hp + 1, c4] = _unpack_pair(w)
        for i in range(2):
            for j in range(2):
                for h in range(_HEADS):
                    q = i * 8 + j * 4 + h
                    acc = _leaky(akv[i, h, 0] + avv[j, h, 0])
                    for c4 in range(1, 4):
                        acc = acc + _leaky(akv[i, h, c4] + avv[j, h, c4])
                    tr_v[pl.ds(q * 16, 16)] = acc
        s = plsc.load_gather(tr_v, [lane * 16])
        for l in range(1, 16):
            s = s + plsc.load_gather(tr_v, [lane * 16 + l])
        s = s * (1.0 / _P)
        sm_v[...] = s
        s_sw = plsc.load_gather(sm_v, [perm_j])
        m = jnp.maximum(s, s_sw)
        e = jnp.exp(s - m)
        sm_v[...] = e
        e_sw = plsc.load_gather(sm_v, [perm_j])
        out_v[pl.ds(g * 16, 16)] = e / (e + e_sw)
        return carry

    lax.fori_loop(0, _GPW, group_body, 0)
    pltpu.sync_copy(out_v, out_hbm.at[pl.ds(wid * (_GPW * 16), _GPW * 16)])


def kernel(x, edge_index, node_idxes, W, att):
    del edge_index
    a_packed = pl.pallas_call(
        _tc_body,
        out_shape=jax.ShapeDtypeStruct((_N, _HEADS), jnp.int32),
    )(x, W, att)
    idx_flat = node_idxes.reshape(_G * _IPG).astype(jnp.int32)
    out = _make_sc_kernel()(a_packed.reshape(_N * _HEADS), idx_flat)
    return out.reshape(_G, 2, 2, _HEADS)


# speedup vs baseline: 1.0685x; 1.0248x over previous
import functools

import jax
import jax.numpy as jnp
from jax import lax
from jax.experimental import pallas as pl
from jax.experimental.pallas import tpu as pltpu
from jax.experimental.pallas import tpu_sc as plsc

_HEADS = 4
_C = 128
_N = 10000
_G = 512
_P = 64
_SLOP = 0.2

_NC = 1
_NS = 16
_NW = _NC * _NS
_GPW = _G // _NW
_IPG = 2 * 2 * _P


def _tc_body(x_ref, w_ref, att_ref, a_ref):
    w = w_ref[...]
    att2 = att_ref[...].reshape(_HEADS, 2 * _C)
    dn = (((1,), (0,)), ((), ()))
    hp = lax.Precision.HIGHEST

    def vrow(h, half):
        avec = att2[h:h + 1, half * _C:(half + 1) * _C]
        wblk = w[h * _C:(h + 1) * _C, :]
        return lax.dot_general(avec, wblk, dn, precision=hp)

    order = [(0, 0), (2, 0), (0, 1), (2, 1), (1, 0), (3, 0), (1, 1), (3, 1)]
    vcat = jnp.concatenate([vrow(h, half) for h, half in order], axis=0)
    dnx = (((1,), (1,)), ((), ()))
    a2 = lax.dot_general(x_ref[...], vcat, dnx)
    lo = lax.bitcast_convert_type(a2[:, :4].astype(jnp.bfloat16), jnp.uint16)
    hi = lax.bitcast_convert_type(a2[:, 4:].astype(jnp.bfloat16), jnp.uint16)
    a_ref[...] = jnp.bitwise_or(
        lo.astype(jnp.int32),
        jnp.left_shift(hi.astype(jnp.int32), 16))


def _leaky(s):
    return jnp.maximum(s, s * _SLOP)


@functools.cache
def _make_sc_kernel():
    mesh = plsc.VectorSubcoreMesh(core_axis_name="c", subcore_axis_name="s", num_cores=_NC)
    return functools.partial(
        pl.kernel,
        mesh=mesh,
        compiler_params=pltpu.CompilerParams(
            needs_layout_passes=False,
            skip_device_barrier=True,
            disable_bounds_checks=True,
            disable_semaphore_checks=True,
        ),
        out_type=jax.ShapeDtypeStruct((_G * 16,), jnp.float32),
        scratch_types=[
            pltpu.VMEM((_N * _HEADS,), jnp.int32),
            pltpu.VMEM((_GPW * _IPG,), jnp.int32),
            pltpu.VMEM((256,), jnp.float32),
            pltpu.VMEM((16,), jnp.float32),
            pltpu.VMEM((_GPW * 16,), jnp.float32),
            pltpu.SemaphoreType.DMA,
        ],
    )(_sc_body)


def _unpack_pair(w):
    lo = plsc.bitcast(jnp.left_shift(w, 16), jnp.float32)
    hi = plsc.bitcast(jnp.bitwise_and(w, jnp.int32(-65536)), jnp.float32)
    return lo, hi


def _sc_body(a_hbm, idx_hbm, out_hbm, a_v, idx_v, tr_v, sm_v, out_v, dma_sem):
    wid = lax.axis_index("s") * _NC + lax.axis_index("c")
    table_cp = pltpu.async_copy(a_hbm, a_v, dma_sem)
    pltpu.sync_copy(idx_hbm.at[pl.ds(wid * (_GPW * _IPG), _GPW * _IPG)], idx_v)
    table_cp.wait()

    lane = lax.iota(jnp.int32, 16)
    perm_j = jnp.bitwise_xor(lane, 4)

    def group_body(g, carry):
        gb = g * _IPG
        kidx = [[idx_v[pl.ds(gb + i * 2 * _P + _P + c4 * 16, 16)] * _HEADS
                 for c4 in range(4)] for i in range(2)]
        vidx = [[idx_v[pl.ds(gb + j * 2 * _P + c4 * 16, 16)] * _HEADS
                 for c4 in range(4)] for j in range(2)]
        akv = {}
        avv = {}
        for c4 in range(4):
            for hp in range(2):
                for i in range(2):
                    w = plsc.load_gather(a_v, [kidx[i][c4] + hp])
                    akv[i, 2 * hp, c4], akv[i, 2 * hp + 1, c4] = _unpack_pair(w)
                for j in range(2):
                    w = plsc.load_gather(a_v, [vidx[j][c4] + (2 + hp)])
                    avv[j, 2 * hp, c4], avv[j, 2 * hp + 1, c4] = _unpack_pair(w)
        for i in range(2):
            for j in range(2):
                for h in range(_HEADS):
                    q = i * 8 + j * 4 + h
                    acc = _leaky(akv[i, h, 0] + avv[j, h, 0])
                    for c4 in range(1, 4):
                        acc = acc + _leaky(akv[i, h, c4] + avv[j, h, c4])
                    tr_v[pl.ds(q * 16, 16)] = acc
        s = plsc.load_gather(tr_v, [lane * 16])
        for l in range(1, 16):
            s = s + plsc.load_gather(tr_v, [lane * 16 + l])
        s = s * (1.0 / _P)
        e = jnp.exp(s)
        sm_v[...] = e
        e_sw = plsc.load_gather(sm_v, [perm_j])
        out_v[pl.ds(g * 16, 16)] = e / (e + e_sw)
        return carry

    lax.fori_loop(0, _GPW, group_body, 0)
    pltpu.sync_copy(out_v, out_hbm.at[pl.ds(wid * (_GPW * 16), _GPW * 16)])


def kernel(x, edge_index, node_idxes, W, att):
    del edge_index
    a_packed = pl.pallas_call(
        _tc_body,
        out_shape=jax.ShapeDtypeStruct((_N, _HEADS), jnp.int32),
    )(x, W, att)
    idx_flat = node_idxes.reshape(_G * _IPG).astype(jnp.int32)
    out = _make_sc_kernel()(a_packed.reshape(_N * _HEADS), idx_flat)
    return out.reshape(_G, 2, 2, _HEADS)


# speedup vs baseline: 1.1091x; 1.0380x over previous
import functools

import jax
import jax.numpy as jnp
from jax import lax
from jax.experimental import pallas as pl
from jax.experimental.pallas import tpu as pltpu
from jax.experimental.pallas import tpu_sc as plsc

_HEADS = 4
_C = 128
_N = 10000
_G = 512
_P = 64
_SLOP = 0.2

_NC = 1
_NS = 16
_NW = _NC * _NS
_GPW = _G // _NW
_IPG = 2 * 2 * _P


def _tc_body(x_ref, w_ref, att_ref, a_ref):
    w = w_ref[...]
    att2 = att_ref[...].reshape(_HEADS, 2 * _C)
    dn = (((1,), (0,)), ((), ()))
    hp = lax.Precision.HIGHEST

    def vrow(h, half):
        avec = att2[h:h + 1, half * _C:(half + 1) * _C]
        wblk = w[h * _C:(h + 1) * _C, :]
        return lax.dot_general(avec, wblk, dn, precision=hp)

    order = [(0, 0), (2, 0), (0, 1), (2, 1), (1, 0), (3, 0), (1, 1), (3, 1)]
    vcat = jnp.concatenate([vrow(h, half) for h, half in order], axis=0)
    dnx = (((1,), (1,)), ((), ()))
    a2 = lax.dot_general(x_ref[...], vcat, dnx)
    lo = lax.bitcast_convert_type(a2[:, :4].astype(jnp.bfloat16), jnp.uint16)
    hi = lax.bitcast_convert_type(a2[:, 4:].astype(jnp.bfloat16), jnp.uint16)
    a_ref[...] = jnp.bitwise_or(
        lo.astype(jnp.int32),
        jnp.left_shift(hi.astype(jnp.int32), 16))


def _leaky(s):
    return jnp.maximum(s, s * _SLOP)


@functools.cache
def _make_sc_kernel():
    mesh = plsc.VectorSubcoreMesh(core_axis_name="c", subcore_axis_name="s", num_cores=_NC)
    return functools.partial(
        pl.kernel,
        mesh=mesh,
        compiler_params=pltpu.CompilerParams(
            needs_layout_passes=False,
            skip_device_barrier=True,
            disable_bounds_checks=True,
            disable_semaphore_checks=True,
        ),
        out_type=jax.ShapeDtypeStruct((_G * 16,), jnp.float32),
        scratch_types=[
            pltpu.VMEM((_N * _HEADS,), jnp.int32),
            pltpu.VMEM((_GPW * _IPG,), jnp.int32),
            pltpu.VMEM((256,), jnp.float32),
            pltpu.VMEM((16,), jnp.float32),
            pltpu.VMEM((_GPW * 16,), jnp.float32),
            pltpu.SemaphoreType.DMA,
        ],
    )(_sc_body)


def _unpack_pair(w):
    lo = plsc.bitcast(jnp.left_shift(w, 16), jnp.float32)
    hi = plsc.bitcast(jnp.bitwise_and(w, jnp.int32(-65536)), jnp.float32)
    return lo, hi


def _sc_body(a_hbm, idx_hbm, out_hbm, a_v, idx_v, tr_v, sm_v, out_v, dma_sem):
    wid = lax.axis_index("s") * _NC + lax.axis_index("c")
    table_cp = pltpu.async_copy(a_hbm, a_v, dma_sem)
    pltpu.sync_copy(idx_hbm.at[pl.ds(wid * (_GPW * _IPG), _GPW * _IPG)], idx_v)
    table_cp.wait()

    lane = lax.iota(jnp.int32, 16)
    perm_j = jnp.bitwise_xor(lane, 4)

    def group_body(g, carry):
        gb = g * _IPG
        kidx = [[idx_v[pl.ds(gb + i * 2 * _P + _P + c4 * 16, 16)] * _HEADS
                 for c4 in range(4)] for i in range(2)]
        vidx = [[idx_v[pl.ds(gb + j * 2 * _P + c4 * 16, 16)] * _HEADS
                 for c4 in range(4)] for j in range(2)]
        akv = {}
        avv = {}
        for c4 in range(4):
            for hp in range(2):
                for i in range(2):
                    w = plsc.load_gather(a_v, [kidx[i][c4] + hp])
                    akv[i, hp, c4] = plsc.bitcast(w, jnp.bfloat16)
                for j in range(2):
                    w = plsc.load_gather(a_v, [vidx[j][c4] + (2 + hp)])
                    avv[j, hp, c4] = plsc.bitcast(w, jnp.bfloat16)
        for i in range(2):
            for j in range(2):
                for hp in range(2):
                    acc = _leaky(akv[i, hp, 0] + avv[j, hp, 0])
                    for c4 in range(1, 4):
                        acc = acc + _leaky(akv[i, hp, c4] + avv[j, hp, c4])
                    lo, hi = _unpack_pair(plsc.bitcast(acc, jnp.int32))
                    tr_v[pl.ds((i * 8 + j * 4 + 2 * hp) * 16, 16)] = lo
                    tr_v[pl.ds((i * 8 + j * 4 + 2 * hp + 1) * 16, 16)] = hi
        s = plsc.load_gather(tr_v, [lane * 16])
        for l in range(1, 16):
            s = s + plsc.load_gather(tr_v, [lane * 16 + l])
        s = s * (1.0 / _P)
        e = jnp.exp(s)
        sm_v[...] = e
        e_sw = plsc.load_gather(sm_v, [perm_j])
        out_v[pl.ds(g * 16, 16)] = e / (e + e_sw)
        return carry

    lax.fori_loop(0, _GPW, group_body, 0)
    pltpu.sync_copy(out_v, out_hbm.at[pl.ds(wid * (_GPW * 16), _GPW * 16)])


def kernel(x, edge_index, node_idxes, W, att):
    del edge_index
    a_packed = pl.pallas_call(
        _tc_body,
        out_shape=jax.ShapeDtypeStruct((_N, _HEADS), jnp.int32),
    )(x, W, att)
    idx_flat = node_idxes.reshape(_G * _IPG).astype(jnp.int32)
    out = _make_sc_kernel()(a_packed.reshape(_N * _HEADS), idx_flat)
    return out.reshape(_G, 2, 2, _HEADS)


# speedup vs baseline: 1.3155x; 1.1862x over previous
import functools

import jax
import jax.numpy as jnp
from jax import lax
from jax.experimental import pallas as pl
from jax.experimental.pallas import tpu as pltpu
from jax.experimental.pallas import tpu_sc as plsc

_HEADS = 4
_C = 128
_N = 10000
_G = 512
_P = 64
_SLOP = 0.2

_NC = 1
_NS = 16
_NW = _NC * _NS
_GPW = _G // _NW
_IPG = 2 * 2 * _P


def _tc_body(x_ref, w_ref, att_ref, a_ref):
    w = w_ref[...]
    att2 = att_ref[...].reshape(_HEADS, 2 * _C)
    dn = (((1,), (0,)), ((), ()))
    hp = lax.Precision.HIGHEST

    def vrow(h, half):
        avec = att2[h:h + 1, half * _C:(half + 1) * _C]
        wblk = w[h * _C:(h + 1) * _C, :]
        return lax.dot_general(avec, wblk, dn, precision=hp)

    order = [(0, 0), (2, 0), (0, 1), (2, 1), (1, 0), (3, 0), (1, 1), (3, 1)]
    vcat = jnp.concatenate([vrow(h, half) for h, half in order], axis=0)
    dnx = (((1,), (1,)), ((), ()))
    a2 = lax.dot_general(vcat, x_ref[...], dnx)
    lo = lax.bitcast_convert_type(a2[:4, :].astype(jnp.bfloat16), jnp.uint16)
    hi = lax.bitcast_convert_type(a2[4:, :].astype(jnp.bfloat16), jnp.uint16)
    a_ref[...] = jnp.bitwise_or(
        lo.astype(jnp.int32),
        jnp.left_shift(hi.astype(jnp.int32), 16))


def _leaky(s):
    return jnp.maximum(s, s * _SLOP)


@functools.cache
def _make_sc_kernel():
    mesh = plsc.VectorSubcoreMesh(core_axis_name="c", subcore_axis_name="s", num_cores=_NC)
    return functools.partial(
        pl.kernel,
        mesh=mesh,
        compiler_params=pltpu.CompilerParams(
            needs_layout_passes=False,
            skip_device_barrier=True,
            disable_bounds_checks=True,
            disable_semaphore_checks=True,
        ),
        out_type=jax.ShapeDtypeStruct((_G * 16,), jnp.float32),
        scratch_types=[
            pltpu.VMEM((_N * _HEADS,), jnp.int32),
            pltpu.VMEM((_GPW * _IPG,), jnp.int32),
            pltpu.VMEM((256,), jnp.float32),
            pltpu.VMEM((16,), jnp.float32),
            pltpu.VMEM((_GPW * 16,), jnp.float32),
            pltpu.SemaphoreType.DMA,
        ],
    )(_sc_body)


def _unpack_pair(w):
    lo = plsc.bitcast(jnp.left_shift(w, 16), jnp.float32)
    hi = plsc.bitcast(jnp.bitwise_and(w, jnp.int32(-65536)), jnp.float32)
    return lo, hi


def _sc_body(a_hbm, idx_hbm, out_hbm, a_v, idx_v, tr_v, sm_v, out_v, dma_sem):
    wid = lax.axis_index("s") * _NC + lax.axis_index("c")
    table_cp = pltpu.async_copy(a_hbm, a_v, dma_sem)
    pltpu.sync_copy(idx_hbm.at[pl.ds(wid * (_GPW * _IPG), _GPW * _IPG)], idx_v)
    table_cp.wait()

    lane = lax.iota(jnp.int32, 16)
    perm_j = jnp.bitwise_xor(lane, 4)

    def group_body(g, carry):
        gb = g * _IPG
        kidx = [[idx_v[pl.ds(gb + i * 2 * _P + _P + c4 * 16, 16)]
                 for c4 in range(4)] for i in range(2)]
        vidx = [[idx_v[pl.ds(gb + j * 2 * _P + c4 * 16, 16)]
                 for c4 in range(4)] for j in range(2)]
        akv = {}
        avv = {}
        for c4 in range(4):
            for hp in range(2):
                for i in range(2):
                    w = plsc.load_gather(a_v, [kidx[i][c4] + hp * _N])
                    akv[i, hp, c4] = plsc.bitcast(w, jnp.bfloat16)
                for j in range(2):
                    w = plsc.load_gather(a_v, [vidx[j][c4] + (2 + hp) * _N])
                    avv[j, hp, c4] = plsc.bitcast(w, jnp.bfloat16)
        for i in range(2):
            for j in range(2):
                for hp in range(2):
                    acc = _leaky(akv[i, hp, 0] + avv[j, hp, 0])
                    for c4 in range(1, 4):
                        acc = acc + _leaky(akv[i, hp, c4] + avv[j, hp, c4])
                    lo, hi = _unpack_pair(plsc.bitcast(acc, jnp.int32))
                    tr_v[pl.ds((i * 8 + j * 4 + 2 * hp) * 16, 16)] = lo
                    tr_v[pl.ds((i * 8 + j * 4 + 2 * hp + 1) * 16, 16)] = hi
        s = plsc.load_gather(tr_v, [lane * 16])
        for l in range(1, 16):
            s = s + plsc.load_gather(tr_v, [lane * 16 + l])
        s = s * (1.0 / _P)
        e = jnp.exp(s)
        sm_v[...] = e
        e_sw = plsc.load_gather(sm_v, [perm_j])
        out_v[pl.ds(g * 16, 16)] = e / (e + e_sw)
        return carry

    lax.fori_loop(0, _GPW, group_body, 0)
    pltpu.sync_copy(out_v, out_hbm.at[pl.ds(wid * (_GPW * 16), _GPW * 16)])


def kernel(x, edge_index, node_idxes, W, att):
    del edge_index
    a_packed = pl.pallas_call(
        _tc_body,
        out_shape=jax.ShapeDtypeStruct((_HEADS, _N), jnp.int32),
    )(x, W, att)
    idx_flat = node_idxes.reshape(_G * _IPG).astype(jnp.int32)
    out = _make_sc_kernel()(a_packed.reshape(_N * _HEADS), idx_flat)
    return out.reshape(_G, 2, 2, _HEADS)
